# baseline (speedup 1.0000x reference)
import functools

import jax
import jax.numpy as jnp
from jax import lax
from jax.experimental import pallas as pl
from jax.experimental.pallas import tpu as pltpu
from jax.experimental.pallas import tpu_sc as plsc

B = 16384
L = 200
D = 32
HALF = 16

NC = 2
NS = 16
NW = NC * NS

ROWS_PER_W = B // NW
CH = 8
NCHUNK = ROWS_PER_W // CH
UNROLL = 8


def _body(x_hbm, table_hbm, bias_hbm, out_hbm,
          idx0, idx1, rows0, rows1, outb, biasb, sem0, sem1):
    wid = lax.axis_index("s") * NC + lax.axis_index("c")
    base_row = wid * ROWS_PER_W

    pltpu.sync_copy(bias_hbm, biasb)
    b_lo = biasb[pl.ds(0, HALF)]
    b_hi = biasb[pl.ds(HALF, HALF)]

    idx_bufs = (idx0, idx1)
    rows_bufs = (rows0, rows1)
    sems = (sem0, sem1)

    def fire(c, b):
        pltpu.sync_copy(x_hbm.at[pl.ds((base_row + c * CH) * L, CH * L)],
                        idx_bufs[b])
        pltpu.async_copy(table_hbm.at[idx_bufs[b]], rows_bufs[b], sems[b])

    def drain(b):
        pltpu.make_async_copy(table_hbm.at[idx_bufs[b]], rows_bufs[b],
                              sems[b]).wait()

    for b in range(2):
        fire(b, b)

    zero = jnp.zeros((HALF,), jnp.float32)

    def do_chunk(cur, b):
        drain(b)
        rows = rows_bufs[b]
        for i in range(CH):
            def red(t, carry):
                s0, s1, s2, s3 = carry
                r = i * L + t * UNROLL
                for u in range(UNROLL):
                    w = rows[r + u, pl.ds(0, HALF)]
                    lo, hi = plsc.unpack(
                        plsc.bitcast(w, jnp.bfloat16),
                        format=plsc.PackFormat.INTERLEAVED)
                    if u % 2 == 0:
                        s0 = s0 + lo
                        s1 = s1 + hi
                    else:
                        s2 = s2 + lo
                        s3 = s3 + hi
                return s0, s1, s2, s3

            s0, s1, s2, s3 = lax.fori_loop(0, L // UNROLL, red,
                                           (zero, zero, zero, zero))
            outb[pl.ds(i * D, HALF)] = s0 + s2 + b_lo
            outb[pl.ds(i * D + HALF, HALF)] = s1 + s3 + b_hi
        pltpu.sync_copy(outb,
                        out_hbm.at[pl.ds((base_row + cur * CH) * D, CH * D)])

        @pl.when(cur + 2 < NCHUNK)
        def _():
            fire(cur + 2, b)

    def outer(g2, carry):
        do_chunk(g2 * 2, 0)
        do_chunk(g2 * 2 + 1, 1)
        return carry

    lax.fori_loop(0, NCHUNK // 2, outer, 0)


V = 1000000
TW = 512
NSLAB_FULL = V // TW
TAIL = V - NSLAB_FULL * TW


XB = 128
XSLAB_W = B // XB // NW


def _transpose_body(tt_hbm, xt_hbm, tail_hbm, out_hbm, xlin_hbm,
                    in0, in1, outb0, outb1, xin, xoutb, tailb, sem0, sem1):
    wid = lax.axis_index("s") * NC + lax.axis_index("c")
    in_bufs = (in0, in1)
    out_bufs = (outb0, outb1)
    sems = (sem0, sem1)
    lane = lax.iota(jnp.int32, 16)

    nslab_w = NSLAB_FULL // NW + 1

    def slab_col(k):
        s = jnp.minimum(k * NW + wid, NSLAB_FULL - 1)
        return s * TW

    def fire(k, b):
        pltpu.async_copy(tt_hbm.at[:, pl.ds(slab_col(k), TW)], in_bufs[b],
                         sems[b])

    def transpose(b):
        perms = [jnp.bitwise_and(lane + j, 15) for j in range(16)]
        obase_lo = [perms[j] * HALF + lane for j in range(16)]
        lane_hi = lane + 16

        def body(t, carry):
            r0 = jnp.full((16,), t * 16, jnp.int32)
            ro = jnp.full((16,), t * 16 * HALF, jnp.int32)
            for h in range(4):
                js = range(h * 4, h * 4 + 4)
                vals = []
                for j in js:
                    c_idx = r0 + perms[j]
                    vals.append(plsc.load_gather(in_bufs[b], [lane, c_idx]))
                    vals.append(plsc.load_gather(in_bufs[b], [lane_hi, c_idx]))
                for k, j in enumerate(js):
                    packed = plsc.bitcast(
                        plsc.pack(vals[2 * k], vals[2 * k + 1],
                                  format=plsc.PackFormat.INTERLEAVED),
                        jnp.int32)
                    plsc.store_scatter(out_bufs[b], [ro + obase_lo[j]], packed)
            return carry

        lax.fori_loop(0, TW // 16, body, 0)

    for b in range(2):
        fire(b, b)

    def step(k, b):
        pltpu.make_async_copy(tt_hbm.at[:, pl.ds(slab_col(k), TW)],
                              in_bufs[b], sems[b]).wait()
        transpose(b)
        pltpu.sync_copy(out_bufs[b],
                        out_hbm.at[pl.ds(slab_col(k) * HALF, TW * HALF)])

        @pl.when(k + 2 < nslab_w)
        def _():
            fire(k + 2, b)

    def outer(k2, carry):
        step(k2 * 2, 0)
        step(k2 * 2 + 1, 1)
        return carry

    lax.fori_loop(0, nslab_w // 2, outer, 0)

    perms = [jnp.bitwise_and(lane + j, 15) for j in range(16)]
    tperms = [jnp.bitwise_and(perms[j], 7) for j in range(16)]
    olane = lane * L

    def x_slab(xs, carry):
        b0 = (xs * NW + wid) * XB
        pltpu.sync_copy(xt_hbm.at[:, pl.ds(b0, XB)], xin)

        def sub(s, carry2):
            k0 = s * HALF
            colv = lane + k0
            base_v = olane + k0 * L
            for l0 in range(0, L - HALF + 1, 16):
                ogrp = base_v + l0
                for j in range(16):
                    v = plsc.load_gather(xin, [perms[j] + l0, colv])
                    plsc.store_scatter(xoutb, [ogrp + perms[j]], v)
            ogrp = base_v + (L - 8)
            for j in range(16):
                m = perms[j] < 8
                v = plsc.load_gather(xin, [tperms[j] + (L - 8), colv], mask=m)
                plsc.store_scatter(xoutb, [ogrp + tperms[j]], v, mask=m)
            return carry2

        lax.fori_loop(0, XB // HALF, sub, 0)
        pltpu.sync_copy(xoutb, xlin_hbm.at[pl.ds(b0 * L, XB * L)])
        return carry

    lax.fori_loop(0, XSLAB_W, x_slab, 0)

    @pl.when(wid == 0)
    def _():
        pltpu.sync_copy(tail_hbm, tailb)
        pltpu.sync_copy(tailb,
                        out_hbm.at[pl.ds(NSLAB_FULL * TW * HALF, TAIL * HALF)])


@jax.jit
def kernel(x, table, bias):
    mesh = plsc.VectorSubcoreMesh(core_axis_name="c", subcore_axis_name="s")
    trans = functools.partial(
        pl.kernel,
        mesh=mesh,
        compiler_params=pltpu.CompilerParams(use_tc_tiling_on_sc=True,
                                             needs_layout_passes=False),
        out_type=(jax.ShapeDtypeStruct((V * HALF,), jnp.int32),
                  jax.ShapeDtypeStruct((B * L,), jnp.int32)),
        scratch_types=[
            pltpu.VMEM((D, TW), jnp.float32),
            pltpu.VMEM((D, TW), jnp.float32),
            pltpu.VMEM((TW * HALF,), jnp.int32),
            pltpu.VMEM((TW * HALF,), jnp.int32),
            pltpu.VMEM((L, XB), jnp.int32),
            pltpu.VMEM((XB * L,), jnp.int32),
            pltpu.VMEM((TAIL * HALF,), jnp.int32),
            pltpu.SemaphoreType.DMA,
            pltpu.SemaphoreType.DMA,
        ],
    )(_transpose_body)
    run = functools.partial(
        pl.kernel,
        mesh=mesh,
        compiler_params=pltpu.CompilerParams(use_tc_tiling_on_sc=False,
                                             needs_layout_passes=False),
        out_type=jax.ShapeDtypeStruct((B * D,), jnp.float32),
        scratch_types=[
            pltpu.VMEM((CH * L,), jnp.int32),
            pltpu.VMEM((CH * L,), jnp.int32),
            pltpu.VMEM((CH * L, HALF), jnp.int32),
            pltpu.VMEM((CH * L, HALF), jnp.int32),
            pltpu.VMEM((CH * D,), jnp.float32),
            pltpu.VMEM((D,), jnp.float32),
            pltpu.SemaphoreType.DMA,
            pltpu.SemaphoreType.DMA,
        ],
    )(_body)
    tail = table[NSLAB_FULL * TW:]
    u = jax.lax.bitcast_convert_type(tail, jnp.uint32)
    rnd = (u + 0x7FFF + ((u >> 16) & 1)) >> 16
    tail_pk = jax.lax.bitcast_convert_type(
        rnd[:, :HALF] | (rnd[:, HALF:] << 16), jnp.int32).reshape(-1)
    table_pk, x_lin = trans(table.T, x.T, tail_pk)
    out = run(x_lin, table_pk.reshape(V, HALF), bias)
    out2d = out.reshape(B, D)
    return (out2d[:, :HALF], out2d[:, HALF:])


# speedup vs baseline: 1.1966x; 1.1966x over previous
import functools

import jax
import jax.numpy as jnp
from jax import lax
from jax.experimental import pallas as pl
from jax.experimental.pallas import tpu as pltpu
from jax.experimental.pallas import tpu_sc as plsc

B = 16384
L = 200
D = 32
HALF = 16

NC = 2
NS = 16
NW = NC * NS

ROWS_PER_W = B // NW
CH = 16
NCHUNK = ROWS_PER_W // CH
UNROLL = 8


def _body(x_hbm, table_hbm, bias_hbm, out_hbm,
          idx0, idx1, rows0, rows1, outb, biasb, sem0, sem1):
    wid = lax.axis_index("s") * NC + lax.axis_index("c")
    base_row = wid * ROWS_PER_W

    pltpu.sync_copy(bias_hbm, biasb)
    b_lo = biasb[pl.ds(0, HALF)]
    b_hi = biasb[pl.ds(HALF, HALF)]

    idx_bufs = (idx0, idx1)
    rows_bufs = (rows0, rows1)
    sems = (sem0, sem1)

    def fire(c, b):
        pltpu.sync_copy(x_hbm.at[pl.ds(base_row + c * CH, CH)], idx_bufs[b])
        for i in range(CH):
            pltpu.async_copy(table_hbm.at[idx_bufs[b].at[i]],
                             rows_bufs[b].at[pl.ds(i * L, L)], sems[b])

    def drain(b):
        for i in range(CH):
            pltpu.make_async_copy(table_hbm.at[idx_bufs[b].at[i]],
                                  rows_bufs[b].at[pl.ds(i * L, L)],
                                  sems[b]).wait()

    for b in range(2):
        fire(b, b)

    zero = jnp.zeros((HALF,), jnp.float32)

    def do_chunk(cur, b):
        drain(b)
        rows = rows_bufs[b]
        for i in range(CH):
            def red(t, carry):
                s0, s1, s2, s3 = carry
                r = i * L + t * UNROLL
                for u in range(UNROLL):
                    w = rows[r + u, pl.ds(0, HALF)]
                    lo, hi = plsc.unpack(
                        plsc.bitcast(w, jnp.bfloat16),
                        format=plsc.PackFormat.INTERLEAVED)
                    if u % 2 == 0:
                        s0 = s0 + lo
                        s1 = s1 + hi
                    else:
                        s2 = s2 + lo
                        s3 = s3 + hi
                return s0, s1, s2, s3

            s0, s1, s2, s3 = lax.fori_loop(0, L // UNROLL, red,
                                           (zero, zero, zero, zero))
            outb[pl.ds(i * D, HALF)] = s0 + s2 + b_lo
            outb[pl.ds(i * D + HALF, HALF)] = s1 + s3 + b_hi
        pltpu.sync_copy(outb,
                        out_hbm.at[pl.ds((base_row + cur * CH) * D, CH * D)])

        @pl.when(cur + 2 < NCHUNK)
        def _():
            fire(cur + 2, b)

    def outer(g2, carry):
        do_chunk(g2 * 2, 0)
        do_chunk(g2 * 2 + 1, 1)
        return carry

    lax.fori_loop(0, NCHUNK // 2, outer, 0)


V = 1000000
TW = 512
NSLAB_FULL = V // TW
TAIL = V - NSLAB_FULL * TW


def _transpose_body(tt_hbm, tail_hbm, out_hbm, in0, in1, outb0, outb1,
                    tailb, sem0, sem1):
    wid = lax.axis_index("s") * NC + lax.axis_index("c")
    in_bufs = (in0, in1)
    out_bufs = (outb0, outb1)
    sems = (sem0, sem1)
    lane = lax.iota(jnp.int32, 16)

    nslab_w = NSLAB_FULL // NW + 1

    def slab_col(k):
        s = jnp.minimum(k * NW + wid, NSLAB_FULL - 1)
        return s * TW

    def fire(k, b):
        pltpu.async_copy(tt_hbm.at[:, pl.ds(slab_col(k), TW)], in_bufs[b],
                         sems[b])

    def transpose(b):
        perms = [jnp.bitwise_and(lane + j, 15) for j in range(16)]
        obase_lo = [perms[j] * HALF + lane for j in range(16)]
        lane_hi = lane + 16

        def body(t, carry):
            r0 = jnp.full((16,), t * 16, jnp.int32)
            ro = jnp.full((16,), t * 16 * HALF, jnp.int32)
            for h in range(4):
                js = range(h * 4, h * 4 + 4)
                vals = []
                for j in js:
                    c_idx = r0 + perms[j]
                    vals.append(plsc.load_gather(in_bufs[b], [lane, c_idx]))
                    vals.append(plsc.load_gather(in_bufs[b], [lane_hi, c_idx]))
                for k, j in enumerate(js):
                    packed = plsc.bitcast(
                        plsc.pack(vals[2 * k], vals[2 * k + 1],
                                  format=plsc.PackFormat.INTERLEAVED),
                        jnp.int32)
                    plsc.store_scatter(out_bufs[b], [ro + obase_lo[j]], packed)
            return carry

        lax.fori_loop(0, TW // 16, body, 0)

    for b in range(2):
        fire(b, b)

    def step(k, b):
        pltpu.make_async_copy(tt_hbm.at[:, pl.ds(slab_col(k), TW)],
                              in_bufs[b], sems[b]).wait()
        transpose(b)
        pltpu.sync_copy(out_bufs[b],
                        out_hbm.at[pl.ds(slab_col(k) * HALF, TW * HALF)])

        @pl.when(k + 2 < nslab_w)
        def _():
            fire(k + 2, b)

    def outer(k2, carry):
        step(k2 * 2, 0)
        step(k2 * 2 + 1, 1)
        return carry

    lax.fori_loop(0, nslab_w // 2, outer, 0)

    @pl.when(wid == 0)
    def _():
        pltpu.sync_copy(tail_hbm, tailb)
        pltpu.sync_copy(tailb,
                        out_hbm.at[pl.ds(NSLAB_FULL * TW * HALF, TAIL * HALF)])


@jax.jit
def kernel(x, table, bias):
    mesh = plsc.VectorSubcoreMesh(core_axis_name="c", subcore_axis_name="s")
    trans = functools.partial(
        pl.kernel,
        mesh=mesh,
        compiler_params=pltpu.CompilerParams(use_tc_tiling_on_sc=True,
                                             needs_layout_passes=False),
        out_type=jax.ShapeDtypeStruct((V * HALF,), jnp.int32),
        scratch_types=[
            pltpu.VMEM((D, TW), jnp.float32),
            pltpu.VMEM((D, TW), jnp.float32),
            pltpu.VMEM((TW * HALF,), jnp.int32),
            pltpu.VMEM((TW * HALF,), jnp.int32),
            pltpu.VMEM((TAIL * HALF,), jnp.int32),
            pltpu.SemaphoreType.DMA,
            pltpu.SemaphoreType.DMA,
        ],
    )(_transpose_body)
    run = functools.partial(
        pl.kernel,
        mesh=mesh,
        compiler_params=pltpu.CompilerParams(use_tc_tiling_on_sc=False,
                                             needs_layout_passes=False),
        out_type=jax.ShapeDtypeStruct((B * D,), jnp.float32),
        scratch_types=[
            pltpu.VMEM((CH, L), jnp.int32),
            pltpu.VMEM((CH, L), jnp.int32),
            pltpu.VMEM((CH * L, HALF), jnp.int32),
            pltpu.VMEM((CH * L, HALF), jnp.int32),
            pltpu.VMEM((CH * D,), jnp.float32),
            pltpu.VMEM((D,), jnp.float32),
            pltpu.SemaphoreType.DMA,
            pltpu.SemaphoreType.DMA,
        ],
    )(_body)
    tail = table[NSLAB_FULL * TW:]
    u = jax.lax.bitcast_convert_type(tail, jnp.uint32)
    rnd = (u + 0x7FFF + ((u >> 16) & 1)) >> 16
    tail_pk = jax.lax.bitcast_convert_type(
        rnd[:, :HALF] | (rnd[:, HALF:] << 16), jnp.int32).reshape(-1)
    table_pk = trans(table.T, tail_pk).reshape(V, HALF)
    out = run(x, table_pk, bias)
    out2d = out.reshape(B, D)
    return (out2d[:, :HALF], out2d[:, HALF:])
